# baseline (speedup 1.0000x reference)
import jax
import jax.numpy as jnp
from jax import lax
from jax.experimental import pallas as pl
from jax.experimental.pallas import tpu as pltpu

_WINDOW = 256
_NQ = 4
_Q = _WINDOW // _NQ
_TAU_INTER = 0.5
_TAU_TEMP = 1.2
_MAX_AGE = 4


def _body(xq_ref, scale_ref, bias_ref, o_ref, ybuf, sig_ref, age_ref,
          sum2_ref, ad2_ref, prev_ref, sem):
    g = pl.program_id(0)
    ng = pl.num_programs(0)
    nw = (ng // _NQ) - 1
    w = lax.div(g, _NQ)
    q = lax.rem(g, _NQ)
    par = lax.rem(w, 2)
    is_data = w < nw

    b = xq_ref.shape[1]
    d = xq_ref.shape[2]
    sc = scale_ref[...]
    bi = bias_ref[...]

    def _copy(win, quarter, parity):
        return pltpu.make_async_copy(
            ybuf.at[parity, pl.ds(quarter * _Q, _Q)],
            o_ref.at[pl.ds(win * _WINDOW + quarter * _Q, _Q)],
            sem.at[parity],
        )

    @pl.when((w >= 2) & is_data)
    def _():
        _copy(w - 2, q, par).wait()

    @pl.when(is_data)
    def _():
        @pl.when(q == 0)
        def _():
            sum2_ref[...] = jnp.zeros((2, b, d), jnp.float32)
            ad2_ref[...] = jnp.zeros((2, b, d), jnp.float32)
            prev_ref[...] = xq_ref[pl.ds(0, 1)]

        def pair_body(i, carry):
            sum2, ad2, prev = carry
            cur2 = xq_ref[pl.ds(2 * i, 2)]
            ybuf[par, pl.ds(q * _Q + 2 * i, 2)] = (
                cur2 * sc[None] + bi[None]
            )
            shifted = jnp.concatenate([prev, cur2[:1]], axis=0)
            return (
                sum2 + cur2,
                ad2 + jnp.abs(cur2 - shifted),
                cur2[1:2],
            )

        carry0 = (sum2_ref[...], ad2_ref[...], prev_ref[...])
        s2, a2, pv = lax.fori_loop(0, _Q // 2, pair_body, carry0)
        sum2_ref[...] = s2
        ad2_ref[...] = a2
        prev_ref[...] = pv

        @pl.when(q == _NQ - 1)
        def _():
            sig = (s2[0] + s2[1]) * (1.0 / _WINDOW)
            ad = a2[0] + a2[1]
            prev_sig = sig_ref[...]
            delta = sig - prev_sig
            d2 = jnp.sum(delta * delta, axis=1, keepdims=True)
            vt = jnp.sum(ad, axis=1, keepdims=True) * (
                1.0 / ((_WINDOW - 1) * d)
            )
            age = age_ref[...]
            refresh = (
                (w == 0)
                | (age >= _MAX_AGE)
                | (d2 > _TAU_INTER * _TAU_INTER * d)
                | (vt > _TAU_TEMP)
            )
            new_sig = jnp.where(refresh, sig, prev_sig)
            sig_ref[...] = new_sig
            age_ref[...] = jnp.where(refresh, 0, age + 1)
            refresh_i = refresh.astype(jnp.int32)
            for bb in range(b):
                @pl.when(refresh_i[bb, 0] == 0)
                def _(bb=bb):
                    row = new_sig[bb : bb + 1] * sc + bi
                    ybuf[par, :, bb, :] = jnp.broadcast_to(
                        row, (_WINDOW, d)
                    )

    @pl.when(w >= 1)
    def _():
        _copy(w - 1, q, 1 - par).start()

    @pl.when(g == ng - 1)
    def _():
        for qq in range(_NQ):
            _copy(nw - 2, qq, lax.rem(nw - 2, 2)).wait()
        for qq in range(_NQ):
            _copy(nw - 1, qq, lax.rem(nw - 1, 2)).wait()


def kernel(x, scale, bias):
    t, b, d = x.shape
    nw = t // _WINDOW
    nq_total = t // _Q
    in_specs = [
        pl.BlockSpec(
            (_Q, b, d),
            lambda g: (jnp.minimum(g, nq_total - 1), 0, 0),
        ),
        pl.BlockSpec((1, d), lambda g: (0, 0)),
        pl.BlockSpec((1, d), lambda g: (0, 0)),
    ]
    out = pl.pallas_call(
        _body,
        grid=((nw + 1) * _NQ,),
        in_specs=in_specs,
        out_specs=pl.BlockSpec(memory_space=pltpu.MemorySpace.HBM),
        out_shape=jax.ShapeDtypeStruct((t, b, d), x.dtype),
        scratch_shapes=[
            pltpu.VMEM((2, _WINDOW, b, d), jnp.float32),
            pltpu.VMEM((b, d), jnp.float32),
            pltpu.VMEM((b, 1), jnp.int32),
            pltpu.VMEM((2, b, d), jnp.float32),
            pltpu.VMEM((2, b, d), jnp.float32),
            pltpu.VMEM((1, b, d), jnp.float32),
            pltpu.SemaphoreType.DMA((2,)),
        ],
    )(x, scale.reshape(1, d), bias.reshape(1, d))
    return out


# speedup vs baseline: 1.6217x; 1.6217x over previous
import jax
import jax.numpy as jnp
from jax.experimental import pallas as pl
from jax.experimental.pallas import tpu as pltpu

_WINDOW = 256
_NQ = 4
_Q = _WINDOW // _NQ
_TAU_INTER = 0.5
_TAU_TEMP = 1.2
_MAX_AGE = 4


def _body(x0, x1, x2, x3, scale_ref, bias_ref, o_ref, ybuf, sig_ref,
          age_ref, sem):
    w = pl.program_id(0)
    nw = pl.num_programs(0)
    par = jax.lax.rem(w, 2)

    def _wait(step, parity):
        for q in range(_NQ):
            pltpu.make_async_copy(
                ybuf.at[parity, pl.ds(q * _Q, _Q)],
                o_ref.at[pl.ds(step * _WINDOW + q * _Q, _Q)],
                sem.at[parity],
            ).wait()

    @pl.when(w >= 2)
    def _():
        _wait(w - 2, par)

    xrefs = [x0, x1, x2, x3]
    b = x0.shape[1]
    d = x0.shape[2]
    sc = scale_ref[...]
    bi = bias_ref[...]

    zero2 = jnp.zeros((2, b, d), jnp.float32)
    sum2 = zero2
    ad2 = zero2
    prev = None
    for qi in range(_NQ):
        xr = xrefs[qi]

        def pair_body(i, carry, qi=qi, xr=xr):
            sum2, ad2, prev = carry
            cur2 = xr[pl.ds(2 * i, 2)]
            ybuf[par, pl.ds(qi * _Q + 2 * i, 2)] = cur2 * sc[None] + bi[None]
            shifted = jnp.concatenate([prev, cur2[:1]], axis=0)
            return (
                sum2 + cur2,
                ad2 + jnp.abs(cur2 - shifted),
                cur2[1:2],
            )

        if qi == 0:
            first = x0[pl.ds(0, 2)]
            ybuf[par, pl.ds(0, 2)] = first * sc[None] + bi[None]
            sum2 = sum2 + first
            ad2 = ad2 + jnp.abs(
                first - jnp.concatenate([first[:1], first[:1]], axis=0)
            )
            prev = first[1:2]
            lo = 1
        else:
            lo = 0
        sum2, ad2, prev = jax.lax.fori_loop(
            lo, _Q // 2, pair_body, (sum2, ad2, prev)
        )

    sig = (sum2[0] + sum2[1]) * (1.0 / _WINDOW)
    ad = ad2[0] + ad2[1]

    prev_sig = sig_ref[...]
    delta = sig - prev_sig
    d2 = jnp.sum(delta * delta, axis=1, keepdims=True)
    vt = jnp.sum(ad, axis=1, keepdims=True) * (
        1.0 / ((_WINDOW - 1) * d)
    )

    age = age_ref[...]
    refresh = (
        (w == 0)
        | (age >= _MAX_AGE)
        | (d2 > _TAU_INTER * _TAU_INTER * d)
        | (vt > _TAU_TEMP)
    )

    new_sig = jnp.where(refresh, sig, prev_sig)
    sig_ref[...] = new_sig
    age_ref[...] = jnp.where(refresh, 0, age + 1)

    refresh_i = refresh.astype(jnp.int32)
    for bb in range(b):
        @pl.when(refresh_i[bb, 0] == 0)
        def _(bb=bb):
            row = new_sig[bb : bb + 1] * sc + bi
            ybuf[par, :, bb, :] = jnp.broadcast_to(row, (_WINDOW, d))

    for q in range(_NQ):
        pltpu.make_async_copy(
            ybuf.at[par, pl.ds(q * _Q, _Q)],
            o_ref.at[pl.ds(w * _WINDOW + q * _Q, _Q)],
            sem.at[par],
        ).start()

    @pl.when(w == nw - 1)
    def _():
        _wait(w - 1, 1 - par)
        _wait(w, par)


def kernel(x, scale, bias):
    t, b, d = x.shape
    nw = t // _WINDOW
    in_specs = [
        pl.BlockSpec((_Q, b, d), (lambda w, qi=qi: (w * _NQ + qi, 0, 0)))
        for qi in range(_NQ)
    ] + [pl.BlockSpec((1, d), lambda w: (0, 0))] * 2
    out = pl.pallas_call(
        _body,
        grid=(nw,),
        in_specs=in_specs,
        out_specs=pl.BlockSpec(memory_space=pltpu.MemorySpace.HBM),
        out_shape=jax.ShapeDtypeStruct((t, b, d), x.dtype),
        scratch_shapes=[
            pltpu.VMEM((2, _WINDOW, b, d), jnp.float32),
            pltpu.VMEM((b, d), jnp.float32),
            pltpu.VMEM((b, 1), jnp.int32),
            pltpu.SemaphoreType.DMA((2,)),
        ],
    )(x, x, x, x, scale.reshape(1, d), bias.reshape(1, d))
    return out
